# baseline (speedup 1.0000x reference)
import functools

import jax
import jax.numpy as jnp
from jax import lax
from jax.experimental import pallas as pl
from jax.experimental.pallas import tpu as pltpu
from jax.experimental.pallas import tpu_sc as plsc

_IN_DIM = 1000000
_D = 64
_EPS = 1e-5
_L = 16
_W = 128
_NB = 4
_UNROLL = 4


def _rsqrt(v):
    bits = lax.bitcast_convert_type(v, jnp.int32)
    y = lax.bitcast_convert_type(jnp.int32(0x5F3759DF) - (bits >> 1),
                                 jnp.float32)
    vh = v * 0.5
    y = y * (1.5 - vh * y * y)
    y = y * (1.5 - vh * y * y)
    return y


def _ln_stats(v0, v1, v2, v3):
    s = (v0 + v1) + (v2 + v3)
    sq = (v0 * v0 + v1 * v1) + (v2 * v2 + v3 * v3)
    mean = jnp.sum(s) * (1.0 / _D)
    var = jnp.sum(sq) * (1.0 / _D) - mean * mean + _EPS
    inv = _rsqrt(jnp.full((_L,), var, jnp.float32))
    return jnp.full((_L,), mean, jnp.float32), inv


_TR_C = 512


def _tc_transpose_table(table):
    V = table.shape[0]
    tabT = table.T

    def body(x_ref, o_ref):
        o_ref[...] = x_ref[...].T

    return pl.pallas_call(
        body,
        grid=((V + _TR_C - 1) // _TR_C,),
        in_specs=[pl.BlockSpec((64, _TR_C), lambda i: (0, i))],
        out_specs=pl.BlockSpec((_TR_C, 64), lambda i: (i, 0)),
        out_shape=jax.ShapeDtypeStruct((V, 64), jnp.float32),
    )(tabT)


def kernel(x0, mask, table, ln_gamma, ln_beta):
    B, F = x0.shape
    N = B * F
    table = _tc_transpose_table(table)
    x0f = x0.reshape(N // _W, _W).astype(jnp.int32)
    mf = mask.reshape(N).astype(jnp.int32)
    gb = jnp.stack([ln_gamma, ln_beta]).astype(jnp.float32)

    info = plsc.get_sparse_core_info()
    nw = info.num_cores * info.num_subcores
    rows_w = N // nw
    n_win = rows_w // _W

    mesh = plsc.VectorSubcoreMesh(core_axis_name="c", subcore_axis_name="s")

    @functools.partial(
        pl.kernel,
        out_type=jax.ShapeDtypeStruct((N, _D), jnp.float32),
        mesh=mesh,
        scratch_types=[
            pltpu.VMEM((n_win, _W), jnp.int32),
            pltpu.VMEM((rows_w,), jnp.int32),
            pltpu.VMEM((_NB, _W, _D), jnp.float32),
            pltpu.VMEM((_NB, _W, _D), jnp.float32),
            pltpu.VMEM((2, _D), jnp.float32),
            pltpu.VMEM((1, _D), jnp.float32),
            pltpu.SemaphoreType.DMA((_NB,)),
            pltpu.SemaphoreType.DMA((_NB,)),
        ],
        compiler_params=pltpu.CompilerParams(needs_layout_passes=False,
                                             use_tc_tiling_on_sc=False),
    )
    def run(x0_hbm, m_hbm, tab_hbm, gb_hbm, out_hbm,
            x0s, ms, rows, obuf, gb_v, pad_v, gsem, osem):
        wid = lax.axis_index("s") * info.num_cores + lax.axis_index("c")
        base = wid * rows_w

        pltpu.sync_copy(gb_hbm, gb_v)
        pltpu.sync_copy(x0_hbm.at[pl.ds(wid * n_win, n_win)], x0s)
        pltpu.sync_copy(m_hbm.at[pl.ds(base, rows_w)], ms)
        pltpu.sync_copy(tab_hbm.at[pl.ds(_IN_DIM, 1)], pad_v)

        gvec = [gb_v[0, pl.ds(j * _L, _L)] for j in range(4)]
        bvec = [gb_v[1, pl.ds(j * _L, _L)] for j in range(4)]
        pvec = [pad_v[0, pl.ds(j * _L, _L)] for j in range(4)]
        pmean, pinv = _ln_stats(*pvec)
        cvec = [(pvec[j] - pmean) * pinv * gvec[j] + bvec[j] for j in range(4)]

        def fire_gather(w, b):
            pltpu.make_async_copy(tab_hbm.at[x0s.at[w]], rows.at[b],
                                  gsem.at[b]).start()

        def wait_gather(w, b):
            pltpu.make_async_copy(tab_hbm.at[x0s.at[w]], rows.at[b],
                                  gsem.at[b]).wait()

        def layer_norm(w, b):
            rb = rows.at[b]
            ob = obuf.at[b]

            @pl.loop(0, _W, step=_UNROLL)
            def _(r0):
                for u in range(_UNROLL):
                    r = r0 + u
                    v = [rb[r, pl.ds(j * _L, _L)] for j in range(4)]
                    mean, inv = _ln_stats(*v)
                    msp = plsc.load_gather(
                        ms, [jnp.full((_L,), w * _W + r, jnp.int32)])
                    keep = msp == 0
                    scale = jnp.where(keep, inv, 0.0)
                    for j in range(4):
                        bias = jnp.where(keep, bvec[j], cvec[j])
                        ob[r, pl.ds(j * _L, _L)] = (
                            (v[j] - mean) * scale * gvec[j] + bias)

        def out_slice(w):
            return out_hbm.at[pl.ds(base + w * _W, _W)]

        for b in range(_NB):
            fire_gather(b, b)

        @pl.loop(0, n_win // _NB)
        def _(i):
            for b in range(_NB):
                w = i * _NB + b
                wait_gather(w, b)

                @pl.when(i > 0)
                def _():
                    pltpu.make_async_copy(obuf.at[b], out_slice(w - _NB),
                                          osem.at[b]).wait()

                layer_norm(w, b)
                pltpu.make_async_copy(obuf.at[b], out_slice(w),
                                      osem.at[b]).start()

                @pl.when(i < n_win // _NB - 1)
                def _():
                    fire_gather(w + _NB, b)

        for b in range(_NB):
            pltpu.make_async_copy(obuf.at[b], out_slice(n_win - _NB + b),
                                  osem.at[b]).wait()

    out = run(x0f, mf, table, gb)
    return out.reshape(B, F, _D)


# speedup vs baseline: 1.3907x; 1.3907x over previous
import functools

import jax
import jax.numpy as jnp
from jax import lax
from jax.experimental import pallas as pl
from jax.experimental.pallas import tpu as pltpu
from jax.experimental.pallas import tpu_sc as plsc

_IN_DIM = 1000000
_D = 64
_EPS = 1e-5
_L = 16
_W = 128
_NB = 4
_UNROLL = 4


def _rsqrt(v):
    bits = lax.bitcast_convert_type(v, jnp.int32)
    y = lax.bitcast_convert_type(jnp.int32(0x5F3759DF) - (bits >> 1),
                                 jnp.float32)
    vh = v * 0.5
    y = y * (1.5 - vh * y * y)
    y = y * (1.5 - vh * y * y)
    return y


def _ln_stats(v0, v1, v2, v3):
    s = (v0 + v1) + (v2 + v3)
    sq = (v0 * v0 + v1 * v1) + (v2 * v2 + v3 * v3)
    mean = jnp.sum(s) * (1.0 / _D)
    var = jnp.sum(sq) * (1.0 / _D) - mean * mean + _EPS
    inv = _rsqrt(jnp.full((_L,), var, jnp.float32))
    return jnp.full((_L,), mean, jnp.float32), inv


def kernel(x0, mask, table, ln_gamma, ln_beta):
    B, F = x0.shape
    x0T = x0.astype(jnp.int32).T
    mT = mask.astype(jnp.int32).T
    gb = jnp.stack([ln_gamma, ln_beta]).astype(jnp.float32)

    info = plsc.get_sparse_core_info()
    nw = info.num_cores * info.num_subcores
    bw = B // nw
    kb = _D // 8
    mg_n = B // _W
    out_words = F * _D * B

    mesh = plsc.VectorSubcoreMesh(core_axis_name="c", subcore_axis_name="s")

    @functools.partial(
        pl.kernel,
        out_type=jax.ShapeDtypeStruct((out_words,), jnp.float32),
        mesh=mesh,
        scratch_types=[
            pltpu.VMEM((F, bw), jnp.int32),
            pltpu.VMEM((F, bw), jnp.int32),
            pltpu.VMEM((_NB, _W, _D), jnp.float32),
            pltpu.VMEM((_NB, kb * 8 * _W), jnp.float32),
            pltpu.VMEM((2, _D), jnp.float32),
            pltpu.VMEM((1, _D), jnp.float32),
            pltpu.SemaphoreType.DMA((_NB,)),
            pltpu.SemaphoreType.DMA((_NB,)),
        ],
        compiler_params=pltpu.CompilerParams(needs_layout_passes=False,
                                             use_tc_tiling_on_sc=False),
    )
    def run(x0_hbm, m_hbm, tab_hbm, gb_hbm, out_hbm,
            xs, ms, rows, obuf, gb_v, pad_v, gsem, osem):
        wid = lax.axis_index("s") * info.num_cores + lax.axis_index("c")
        b0 = wid * bw

        pltpu.sync_copy(gb_hbm, gb_v)
        pltpu.sync_copy(x0_hbm.at[:, pl.ds(b0, bw)], xs)
        pltpu.sync_copy(m_hbm.at[:, pl.ds(b0, bw)], ms)
        pltpu.sync_copy(tab_hbm.at[pl.ds(_IN_DIM, 1)], pad_v)

        gvec = [gb_v[0, pl.ds(j * _L, _L)] for j in range(4)]
        bvec = [gb_v[1, pl.ds(j * _L, _L)] for j in range(4)]
        pvec = [pad_v[0, pl.ds(j * _L, _L)] for j in range(4)]
        pmean, pinv = _ln_stats(*pvec)
        cvec = [(pvec[j] - pmean) * pinv * gvec[j] + bvec[j] for j in range(4)]

        ib = lax.iota(jnp.int32, _L)
        caddr = [(jnp.int32(2 * j) + (ib >> 3)) * 1024 + (ib & 7) * 128
                 for j in range(4)]

        def fire_gather(f, m):
            pltpu.make_async_copy(tab_hbm.at[xs.at[f, pl.ds(m * _W, _W)]],
                                  rows.at[m], gsem.at[m]).start()

        def wait_gather(f, m):
            pltpu.make_async_copy(tab_hbm.at[xs.at[f, pl.ds(m * _W, _W)]],
                                  rows.at[m], gsem.at[m]).wait()

        def out_chunk(f, k, m):
            off = ((f * kb + k) * mg_n + wid * _NB + m) * (8 * _W)
            return out_hbm.at[pl.ds(off, 8 * _W)]

        def fire_out(f, m):
            for k in range(kb):
                pltpu.make_async_copy(obuf.at[m, pl.ds(k * 8 * _W, 8 * _W)],
                                      out_chunk(f, k, m), osem.at[m]).start()

        def wait_out(f, m):
            for k in range(kb):
                pltpu.make_async_copy(obuf.at[m, pl.ds(k * 8 * _W, 8 * _W)],
                                      out_chunk(f, k, m), osem.at[m]).wait()

        def layer_norm(f, m):
            rb = rows.at[m]
            ob = obuf.at[m]

            @pl.loop(0, _W, step=_UNROLL)
            def _(r0):
                for u in range(_UNROLL):
                    r = r0 + u
                    v = [rb[r, pl.ds(j * _L, _L)] for j in range(4)]
                    mean, inv = _ln_stats(*v)
                    msp = plsc.load_gather(
                        ms, [jnp.full((_L,), f, jnp.int32),
                             jnp.full((_L,), m * _W + r, jnp.int32)])
                    keep = msp == 0
                    scale = jnp.where(keep, inv, 0.0)
                    for j in range(4):
                        bias = jnp.where(keep, bvec[j], cvec[j])
                        val = (v[j] - mean) * scale * gvec[j] + bias
                        plsc.store_scatter(ob, [caddr[j] + r], val)

        for m in range(_NB):
            fire_gather(0, m)

        @pl.loop(0, F)
        def _(f):
            for m in range(_NB):
                wait_gather(f, m)

                @pl.when(f > 0)
                def _():
                    wait_out(f - 1, m)

                layer_norm(f, m)
                fire_out(f, m)

                @pl.when(f < F - 1)
                def _():
                    fire_gather(f + 1, m)

        for m in range(_NB):
            wait_out(F - 1, m)

    out_flat = run(x0T, mT, table, gb)
    out5 = out_flat.reshape(F, kb, mg_n, 8, _W)
    return out5.transpose(2, 4, 0, 1, 3).reshape(B, F, _D)


# speedup vs baseline: 1.7869x; 1.2849x over previous
import functools

import jax
import jax.numpy as jnp
from jax import lax
from jax.experimental import pallas as pl
from jax.experimental.pallas import tpu as pltpu
from jax.experimental.pallas import tpu_sc as plsc

_IN_DIM = 1000000
_D = 64
_EPS = 1e-5
_L = 16
_W = 128
_NB = 4
_UNROLL = 4


def _rsqrt(v):
    bits = lax.bitcast_convert_type(v, jnp.int32)
    y = lax.bitcast_convert_type(jnp.int32(0x5F3759DF) - (bits >> 1),
                                 jnp.float32)
    vh = v * 0.5
    y = y * (1.5 - vh * y * y)
    y = y * (1.5 - vh * y * y)
    return y


def _ln_stats(v0, v1, v2, v3):
    s = (v0 + v1) + (v2 + v3)
    sq = (v0 * v0 + v1 * v1) + (v2 * v2 + v3 * v3)
    mean = jnp.sum(s) * (1.0 / _D)
    var = jnp.sum(sq) * (1.0 / _D) - mean * mean + _EPS
    inv = _rsqrt(jnp.full((_L,), var, jnp.float32))
    return jnp.full((_L,), mean, jnp.float32), inv


def kernel(x0, mask, table, ln_gamma, ln_beta):
    B, F = x0.shape
    N = B * F
    x0T = x0.astype(jnp.int32).T
    mT = mask.astype(jnp.int32).T
    gb = jnp.stack([ln_gamma, ln_beta]).astype(jnp.float32)

    info = plsc.get_sparse_core_info()
    nw = info.num_cores * info.num_subcores
    bw = B // nw
    rows_w = bw * F
    n_win = rows_w // _W

    mesh = plsc.VectorSubcoreMesh(core_axis_name="c", subcore_axis_name="s")

    @functools.partial(
        pl.kernel,
        out_type=jax.ShapeDtypeStruct((N, _D), jnp.float32),
        mesh=mesh,
        scratch_types=[
            pltpu.VMEM((F, bw), jnp.int32),
            pltpu.VMEM((F, bw), jnp.int32),
            pltpu.VMEM((rows_w,), jnp.int32),
            pltpu.VMEM((rows_w,), jnp.int32),
            pltpu.VMEM((_NB, _W, _D), jnp.float32),
            pltpu.VMEM((_NB, _W, _D), jnp.float32),
            pltpu.VMEM((2, _D), jnp.float32),
            pltpu.VMEM((1, _D), jnp.float32),
            pltpu.SemaphoreType.DMA((_NB,)),
            pltpu.SemaphoreType.DMA((_NB,)),
        ],
        compiler_params=pltpu.CompilerParams(needs_layout_passes=False,
                                             use_tc_tiling_on_sc=False),
    )
    def run(x0_hbm, m_hbm, tab_hbm, gb_hbm, out_hbm,
            xs, ms, idxf, mf, rows, obuf, gb_v, pad_v, gsem, osem):
        wid = lax.axis_index("s") * info.num_cores + lax.axis_index("c")
        b0 = wid * bw
        base = wid * rows_w

        pltpu.sync_copy(gb_hbm, gb_v)
        pltpu.sync_copy(x0_hbm.at[:, pl.ds(b0, bw)], xs)
        pltpu.sync_copy(m_hbm.at[:, pl.ds(b0, bw)], ms)
        pltpu.sync_copy(tab_hbm.at[pl.ds(_IN_DIM, 1)], pad_v)

        gvec = [gb_v[0, pl.ds(j * _L, _L)] for j in range(4)]
        bvec = [gb_v[1, pl.ds(j * _L, _L)] for j in range(4)]
        pvec = [pad_v[0, pl.ds(j * _L, _L)] for j in range(4)]
        pmean, pinv = _ln_stats(*pvec)
        cvec = [(pvec[j] - pmean) * pinv * gvec[j] + bvec[j] for j in range(4)]

        ib = lax.iota(jnp.int32, _L)

        @pl.loop(0, F)
        def _(f):
            @pl.loop(0, bw, step=_L)
            def _(bb):
                addr = (jnp.full((_L,), bb, jnp.int32) + ib) * F + f
                plsc.store_scatter(idxf, [addr], xs[f, pl.ds(bb, _L)])
                plsc.store_scatter(mf, [addr], ms[f, pl.ds(bb, _L)])

        def fire_gather(w, b):
            pltpu.make_async_copy(tab_hbm.at[idxf.at[pl.ds(w * _W, _W)]],
                                  rows.at[b], gsem.at[b]).start()

        def wait_gather(w, b):
            pltpu.make_async_copy(tab_hbm.at[idxf.at[pl.ds(w * _W, _W)]],
                                  rows.at[b], gsem.at[b]).wait()

        def out_slice(w):
            return out_hbm.at[pl.ds(base + w * _W, _W)]

        def layer_norm(w, b):
            rb = rows.at[b]
            ob = obuf.at[b]

            @pl.loop(0, _W, step=_UNROLL)
            def _(r0):
                for u in range(_UNROLL):
                    r = r0 + u
                    v = [rb[r, pl.ds(j * _L, _L)] for j in range(4)]
                    mean, inv = _ln_stats(*v)
                    msp = plsc.load_gather(
                        mf, [jnp.full((_L,), w * _W + r, jnp.int32)])
                    keep = msp == 0
                    scale = jnp.where(keep, inv, 0.0)
                    for j in range(4):
                        bias = jnp.where(keep, bvec[j], cvec[j])
                        ob[r, pl.ds(j * _L, _L)] = (
                            (v[j] - mean) * scale * gvec[j] + bias)

        for b in range(_NB):
            fire_gather(b, b)

        @pl.loop(0, n_win // _NB)
        def _(i):
            for b in range(_NB):
                w = i * _NB + b
                wait_gather(w, b)

                @pl.when(i > 0)
                def _():
                    pltpu.make_async_copy(obuf.at[b], out_slice(w - _NB),
                                          osem.at[b]).wait()

                layer_norm(w, b)
                pltpu.make_async_copy(obuf.at[b], out_slice(w),
                                      osem.at[b]).start()

                @pl.when(i < n_win // _NB - 1)
                def _():
                    fire_gather(w + _NB, b)

        for b in range(_NB):
            pltpu.make_async_copy(obuf.at[b], out_slice(n_win - _NB + b),
                                  osem.at[b]).wait()

    out = run(x0T, mT, table, gb)
    return out.reshape(B, F, _D)
